# baseline (speedup 1.0000x reference)
import jax
import jax.numpy as jnp
from jax.experimental import pallas as pl
from jax.experimental.pallas import tpu as pltpu


def _copy_step(x_ref, o_ref):
    o_ref[...] = x_ref[...]


def kernel(x, w1, b1, w2, b2):
    B, C, H, W = x.shape
    N = C * H * W
    x2 = x.reshape(B, N)
    bt = 8
    steps = B // bt
    out = pl.pallas_call(
        _copy_step,
        out_shape=jax.ShapeDtypeStruct((B, N), x.dtype),
        grid=(steps,),
        in_specs=[pl.BlockSpec((bt, N), lambda i: (i, 0))],
        out_specs=pl.BlockSpec((bt, N), lambda i: (i, 0)),
        compiler_params=pltpu.CompilerParams(
            dimension_semantics=("parallel",),
            vmem_limit_bytes=48 * 1024 * 1024,
        ),
    )(x2)
    return out.reshape(B, C, H, W)


# speedup vs baseline: 9.0247x; 9.0247x over previous
import functools

import jax
import jax.numpy as jnp
from jax.experimental import pallas as pl
from jax.experimental.pallas import tpu as pltpu


def _se_step(x_ref, w1t_ref, b1_ref, w2t_ref, b2_ref, o_ref, *, inv_hw):
    x = x_ref[...]
    y = jnp.sum(x, axis=0) * inv_hw
    h = jnp.dot(y, w1t_ref[...], preferred_element_type=jnp.float32)
    h = jnp.maximum(h + b1_ref[...], 0.0)
    z = jnp.dot(h, w2t_ref[...], preferred_element_type=jnp.float32)
    z = z + b2_ref[...]
    s = jnp.clip(z + 3.0, 0.0, 6.0) * (1.0 / 6.0)
    o_ref[...] = x * s[None]


def kernel(x, w1, b1, w2, b2):
    B, C, H, W = x.shape
    hidden = w1.shape[0]
    HW = H * W

    xt = jnp.transpose(x, (2, 3, 0, 1)).reshape(HW, B, C)

    w1t = jnp.transpose(w1).astype(jnp.float32)
    w2t = jnp.transpose(w2).astype(jnp.float32)
    b1r = b1.reshape(1, hidden).astype(jnp.float32)
    b2r = b2.reshape(1, C).astype(jnp.float32)

    bt = 8
    while B % bt:
        bt //= 2
    steps = B // bt

    body = functools.partial(_se_step, inv_hw=1.0 / float(HW))

    itemsize = jnp.dtype(x.dtype).itemsize
    cost = pl.CostEstimate(
        flops=2 * B * C * HW + 4 * B * C * hidden,
        transcendentals=0,
        bytes_accessed=2 * B * C * HW * itemsize,
    )

    out_t = pl.pallas_call(
        body,
        out_shape=jax.ShapeDtypeStruct((HW, B, C), x.dtype),
        grid=(steps,),
        in_specs=[
            pl.BlockSpec((HW, bt, C), lambda i: (0, i, 0)),
            pl.BlockSpec((C, hidden), lambda i: (0, 0)),
            pl.BlockSpec((1, hidden), lambda i: (0, 0)),
            pl.BlockSpec((hidden, C), lambda i: (0, 0)),
            pl.BlockSpec((1, C), lambda i: (0, 0)),
        ],
        out_specs=pl.BlockSpec((HW, bt, C), lambda i: (0, i, 0)),
        compiler_params=pltpu.CompilerParams(
            dimension_semantics=("parallel",),
            vmem_limit_bytes=48 * 1024 * 1024,
        ),
        cost_estimate=cost,
    )(xt, w1t, b1r, w2t, b2r)

    return jnp.transpose(out_t.reshape(H, W, B, C), (2, 3, 0, 1))


# speedup vs baseline: 9.5971x; 1.0634x over previous
import functools

import jax
import jax.numpy as jnp
from jax.experimental import pallas as pl
from jax.experimental.pallas import tpu as pltpu


def _se_step(x_ref, w1t_ref, b1_ref, w2t_ref, b2_ref, o_ref, *, inv_hw):
    x = x_ref[...]
    y = jnp.sum(x, axis=0) * inv_hw
    h = jnp.dot(y, w1t_ref[...], preferred_element_type=jnp.float32)
    h = jnp.maximum(h + b1_ref[...], 0.0)
    z = jnp.dot(h, w2t_ref[...], preferred_element_type=jnp.float32)
    z = z + b2_ref[...]
    s = jnp.clip(z + 3.0, 0.0, 6.0) * (1.0 / 6.0)
    o_ref[...] = x * s[None]


def kernel(x, w1, b1, w2, b2):
    B, C, H, W = x.shape
    hidden = w1.shape[0]
    HW = H * W

    xt = jnp.transpose(x, (2, 3, 0, 1)).reshape(HW, B, C)

    w1t = jnp.transpose(w1).astype(jnp.float32)
    w2t = jnp.transpose(w2).astype(jnp.float32)
    b1r = b1.reshape(1, hidden).astype(jnp.float32)
    b2r = b2.reshape(1, C).astype(jnp.float32)

    bt = 16
    while B % bt:
        bt //= 2
    steps = B // bt

    body = functools.partial(_se_step, inv_hw=1.0 / float(HW))

    itemsize = jnp.dtype(x.dtype).itemsize
    cost = pl.CostEstimate(
        flops=2 * B * C * HW + 4 * B * C * hidden,
        transcendentals=0,
        bytes_accessed=2 * B * C * HW * itemsize,
    )

    out_t = pl.pallas_call(
        body,
        out_shape=jax.ShapeDtypeStruct((HW, B, C), x.dtype),
        grid=(steps,),
        in_specs=[
            pl.BlockSpec((HW, bt, C), lambda i: (0, i, 0)),
            pl.BlockSpec((C, hidden), lambda i: (0, 0)),
            pl.BlockSpec((1, hidden), lambda i: (0, 0)),
            pl.BlockSpec((hidden, C), lambda i: (0, 0)),
            pl.BlockSpec((1, C), lambda i: (0, 0)),
        ],
        out_specs=pl.BlockSpec((HW, bt, C), lambda i: (0, i, 0)),
        compiler_params=pltpu.CompilerParams(
            dimension_semantics=("parallel",),
            vmem_limit_bytes=56 * 1024 * 1024,
        ),
        cost_estimate=cost,
    )(xt, w1t, b1r, w2t, b2r)

    return jnp.transpose(out_t.reshape(H, W, B, C), (2, 3, 0, 1))


# speedup vs baseline: 9.6109x; 1.0014x over previous
import functools

import jax
import jax.numpy as jnp
from jax.experimental import pallas as pl
from jax.experimental.pallas import tpu as pltpu


def _se_step(x_ref, w1t_ref, b1_ref, w2t_ref, b2_ref, o_ref, *, inv_hw):
    x = x_ref[...]
    y = jnp.sum(x, axis=0) * inv_hw
    h = jnp.dot(y, w1t_ref[...], preferred_element_type=jnp.float32)
    h = jnp.maximum(h + b1_ref[...], 0.0)
    z = jnp.dot(h, w2t_ref[...], preferred_element_type=jnp.float32)
    z = z + b2_ref[...]
    s = jnp.clip(z + 3.0, 0.0, 6.0) * (1.0 / 6.0)
    o_ref[...] = x * s[None]


def kernel(x, w1, b1, w2, b2):
    B, C, H, W = x.shape
    hidden = w1.shape[0]
    HW = H * W

    xt = jnp.transpose(x, (2, 3, 0, 1)).reshape(HW, B, C)

    w1t = jnp.transpose(w1).astype(jnp.float32)
    w2t = jnp.transpose(w2).astype(jnp.float32)
    b1r = b1.reshape(1, hidden).astype(jnp.float32)
    b2r = b2.reshape(1, C).astype(jnp.float32)

    bt = B
    for cand_bt in (16, 8):
        if B % cand_bt == 0:
            bt = cand_bt
            break
    steps = B // bt

    body = functools.partial(_se_step, inv_hw=1.0 / float(HW))

    itemsize = jnp.dtype(x.dtype).itemsize
    cost = pl.CostEstimate(
        flops=2 * B * C * HW + 4 * B * C * hidden,
        transcendentals=0,
        bytes_accessed=2 * B * C * HW * itemsize,
    )

    out_t = pl.pallas_call(
        body,
        out_shape=jax.ShapeDtypeStruct((HW, B, C), x.dtype),
        grid=(steps,),
        in_specs=[
            pl.BlockSpec((HW, bt, C), lambda i: (0, i, 0)),
            pl.BlockSpec((C, hidden), lambda i: (0, 0)),
            pl.BlockSpec((1, hidden), lambda i: (0, 0)),
            pl.BlockSpec((hidden, C), lambda i: (0, 0)),
            pl.BlockSpec((1, C), lambda i: (0, 0)),
        ],
        out_specs=pl.BlockSpec((HW, bt, C), lambda i: (0, i, 0)),
        compiler_params=pltpu.CompilerParams(
            dimension_semantics=("parallel",),
            vmem_limit_bytes=56 * 1024 * 1024,
        ),
        cost_estimate=cost,
    )(xt, w1t, b1r, w2t, b2r)

    return jnp.transpose(out_t.reshape(H, W, B, C), (2, 3, 0, 1))
